# Initial kernel scaffold; baseline (speedup 1.0000x reference)
import jax
import jax.numpy as jnp
from jax.experimental import pallas as pl


def kernel(z, elec_table, W):
    raise NotImplementedError("write your pallas kernel here")


# speedup vs baseline: 15.2395x; 15.2395x over previous
import functools

import jax
import jax.numpy as jnp
from jax import lax
from jax.experimental import pallas as pl
from jax.experimental.pallas import tpu as pltpu
from jax.experimental.pallas import tpu_sc as plsc

N_ORB = 37
EMBED = 32
ROW = N_ORB * EMBED
NZROWS = 97
W_ROWS = 16
W_FLAT = N_ORB * W_ROWS * EMBED
ELEC_PAD = 3600
N_NODES = 50000

_info = plsc.get_sparse_core_info()
NC = _info.num_cores
NS = _info.num_subcores
NW = NC * NS

B_PER_W = 1560
CHUNK = 40
NCHUNK = B_PER_W // CHUNK
TAIL_BASE = NW * B_PER_W
TAIL_CHUNKS = (N_NODES - TAIL_BASE) // CHUNK
ROWS_PER_SUB = 7


def _sc_body(z_hbm, elec_hbm, w_hbm, out_hbm,
             t_sh, w_v, e_v, trow_v, zloc_v, buf_v, sem):
    c = lax.axis_index("c")
    s = lax.axis_index("s")
    wid = s * NC + c

    pltpu.sync_copy(w_hbm, w_v)
    pltpu.sync_copy(elec_hbm, e_v)
    lanes = lax.iota(jnp.int32, 16)

    def build_row(r, carry):
        zv = s * ROWS_PER_SUB + r

        @pl.when(zv < NZROWS)
        def _():
            def body(i, carry2):
                f0 = i * 16
                f = f0 + lanes
                o = lax.shift_right_logical(f, 5)
                d = lax.bitwise_and(f, 31)
                e = plsc.load_gather(e_v, [zv * N_ORB + o])
                val = plsc.load_gather(w_v, [o * (W_ROWS * EMBED) + e * EMBED + d])
                val = jnp.where(e == 0, 0.0, val)
                trow_v[pl.ds(f0, 16)] = val
                return carry2

            lax.fori_loop(0, ROW // 16, body, 0)
            pltpu.sync_copy(trow_v, t_sh.at[zv])

        return carry

    lax.fori_loop(0, ROWS_PER_SUB, build_row, 0)
    plsc.subcore_barrier()

    base = wid * B_PER_W
    pltpu.sync_copy(z_hbm.at[pl.ds(base, B_PER_W)], zloc_v)

    def do_chunk(g, carry):
        idx = zloc_v.at[pl.ds(g * CHUNK, CHUNK)]
        pltpu.async_copy(t_sh.at[idx], buf_v, sem).wait()
        pltpu.sync_copy(buf_v, out_hbm.at[pl.ds(base + g * CHUNK, CHUNK)])
        return carry

    lax.fori_loop(0, NCHUNK, do_chunk, 0)

    @pl.when(wid < TAIL_CHUNKS)
    def _():
        tb = TAIL_BASE + wid * CHUNK
        pltpu.sync_copy(z_hbm.at[pl.ds(tb, CHUNK)], zloc_v.at[pl.ds(0, CHUNK)])
        idx = zloc_v.at[pl.ds(0, CHUNK)]
        pltpu.async_copy(t_sh.at[idx], buf_v, sem).wait()
        pltpu.sync_copy(buf_v, out_hbm.at[pl.ds(tb, CHUNK)])


@jax.jit
def _run(z, elec_flat, w_flat):
    mesh = plsc.VectorSubcoreMesh(core_axis_name="c", subcore_axis_name="s")
    f = pl.kernel(
        _sc_body,
        out_type=jax.ShapeDtypeStruct((N_NODES, ROW), jnp.float32),
        mesh=mesh,
        compiler_params=pltpu.CompilerParams(
            needs_layout_passes=False, use_tc_tiling_on_sc=False),
        scratch_types=[
            pltpu.VMEM_SHARED((NZROWS, ROW), jnp.float32),
            pltpu.VMEM((W_FLAT,), jnp.float32),
            pltpu.VMEM((ELEC_PAD,), jnp.int32),
            pltpu.VMEM((ROW,), jnp.float32),
            pltpu.VMEM((B_PER_W,), jnp.int32),
            pltpu.VMEM((CHUNK, ROW), jnp.float32),
            pltpu.SemaphoreType.DMA,
        ],
    )
    return f(z, elec_flat, w_flat)


def kernel(z, elec_table, W):
    elec_flat = jnp.zeros((ELEC_PAD,), jnp.int32).at[: NZROWS * N_ORB].set(
        elec_table.reshape(-1))
    out = _run(z, elec_flat, W.reshape(-1))
    return out.reshape(N_NODES, N_ORB, EMBED)


# speedup vs baseline: 15.7712x; 1.0349x over previous
import functools

import jax
import jax.numpy as jnp
from jax import lax
from jax.experimental import pallas as pl
from jax.experimental.pallas import tpu as pltpu
from jax.experimental.pallas import tpu_sc as plsc

N_ORB = 37
EMBED = 32
ROW = N_ORB * EMBED
NZROWS = 97
W_ROWS = 16
W_FLAT = N_ORB * W_ROWS * EMBED
ELEC_PAD = 3600
N_NODES = 50000

_info = plsc.get_sparse_core_info()
NC = _info.num_cores
NS = _info.num_subcores
NW = NC * NS

B_PER_W = 1560
CHUNK = 40
NCHUNK = B_PER_W // CHUNK
TAIL_BASE = NW * B_PER_W
TAIL_CHUNKS = (N_NODES - TAIL_BASE) // CHUNK
ROWS_PER_SUB = 7


def _sc_body(z_hbm, elec_hbm, w_hbm, out_hbm,
             t_sh, w_v, e_v, trow_v, zloc_v, buf0_v, buf1_v,
             gsem0, gsem1, wsem0, wsem1):
    c = lax.axis_index("c")
    s = lax.axis_index("s")
    wid = s * NC + c

    pltpu.sync_copy(w_hbm, w_v)
    pltpu.sync_copy(elec_hbm, e_v)
    lanes = lax.iota(jnp.int32, 16)

    def build_row(r, carry):
        zv = s * ROWS_PER_SUB + r

        @pl.when(zv < NZROWS)
        def _():
            def body(i, carry2):
                f0 = i * 16
                f = f0 + lanes
                o = lax.shift_right_logical(f, 5)
                d = lax.bitwise_and(f, 31)
                e = plsc.load_gather(e_v, [zv * N_ORB + o])
                val = plsc.load_gather(w_v, [o * (W_ROWS * EMBED) + e * EMBED + d])
                val = jnp.where(e == 0, 0.0, val)
                trow_v[pl.ds(f0, 16)] = val
                return carry2

            lax.fori_loop(0, ROW // 16, body, 0)
            pltpu.sync_copy(trow_v, t_sh.at[zv])

        return carry

    lax.fori_loop(0, ROWS_PER_SUB, build_row, 0)
    plsc.subcore_barrier()

    base = wid * B_PER_W
    pltpu.sync_copy(z_hbm.at[pl.ds(base, B_PER_W)], zloc_v)

    def g_start(g, buf, sem):
        pltpu.async_copy(t_sh.at[zloc_v.at[pl.ds(g * CHUNK, CHUNK)]], buf, sem)

    def g_wait(buf, sem):
        pltpu.make_async_copy(
            t_sh.at[zloc_v.at[pl.ds(0, CHUNK)]], buf, sem).wait()

    def w_start(g, buf, sem):
        pltpu.async_copy(buf, out_hbm.at[pl.ds(base + g * CHUNK, CHUNK)], sem)

    def w_wait(buf, sem):
        pltpu.make_async_copy(buf, out_hbm.at[pl.ds(base, CHUNK)], sem).wait()

    g_start(0, buf0_v, gsem0)

    def pipe(p, carry):
        a = 2 * p
        b = a + 1
        g_wait(buf0_v, gsem0)

        @pl.when(b < NCHUNK)
        def _():
            @pl.when(p > 0)
            def _():
                w_wait(buf1_v, wsem1)
            g_start(b, buf1_v, gsem1)

        w_start(a, buf0_v, wsem0)

        @pl.when(b < NCHUNK)
        def _():
            g_wait(buf1_v, gsem1)

            @pl.when(b + 1 < NCHUNK)
            def _():
                w_wait(buf0_v, wsem0)
                g_start(b + 1, buf0_v, gsem0)

            w_start(b, buf1_v, wsem1)

        return carry

    lax.fori_loop(0, (NCHUNK + 1) // 2, pipe, 0)

    w_wait(buf0_v, wsem0)
    w_wait(buf1_v, wsem1)

    @pl.when(wid < TAIL_CHUNKS)
    def _():
        tb = TAIL_BASE + wid * CHUNK
        pltpu.sync_copy(z_hbm.at[pl.ds(tb, CHUNK)], zloc_v.at[pl.ds(0, CHUNK)])
        idx = zloc_v.at[pl.ds(0, CHUNK)]
        pltpu.async_copy(t_sh.at[idx], buf0_v, gsem0).wait()
        pltpu.sync_copy(buf0_v, out_hbm.at[pl.ds(tb, CHUNK)])


@jax.jit
def _run(z, elec_flat, w_flat):
    mesh = plsc.VectorSubcoreMesh(core_axis_name="c", subcore_axis_name="s")
    f = pl.kernel(
        _sc_body,
        out_type=jax.ShapeDtypeStruct((N_NODES, ROW), jnp.float32),
        mesh=mesh,
        compiler_params=pltpu.CompilerParams(
            needs_layout_passes=False, use_tc_tiling_on_sc=False),
        scratch_types=[
            pltpu.VMEM_SHARED((NZROWS, ROW), jnp.float32),
            pltpu.VMEM((W_FLAT,), jnp.float32),
            pltpu.VMEM((ELEC_PAD,), jnp.int32),
            pltpu.VMEM((ROW,), jnp.float32),
            pltpu.VMEM((B_PER_W,), jnp.int32),
            pltpu.VMEM((CHUNK, ROW), jnp.float32),
            pltpu.VMEM((CHUNK, ROW), jnp.float32),
            pltpu.SemaphoreType.DMA,
            pltpu.SemaphoreType.DMA,
            pltpu.SemaphoreType.DMA,
            pltpu.SemaphoreType.DMA,
        ],
    )
    return f(z, elec_flat, w_flat)


def kernel(z, elec_table, W):
    elec_flat = jnp.zeros((ELEC_PAD,), jnp.int32).at[: NZROWS * N_ORB].set(
        elec_table.reshape(-1))
    out = _run(z, elec_flat, W.reshape(-1))
    return out.reshape(N_NODES, N_ORB, EMBED)


# speedup vs baseline: 20.4252x; 1.2951x over previous
import functools

import jax
import jax.numpy as jnp
from jax import lax
from jax.experimental import pallas as pl
from jax.experimental.pallas import tpu as pltpu
from jax.experimental.pallas import tpu_sc as plsc

N_ORB = 37
EMBED = 32
ROW = N_ORB * EMBED
NZROWS = 97
W_ROWS = 16
W_FLAT = N_ORB * W_ROWS * EMBED
ELEC_PAD = 3600
N_NODES = 50000

_info = plsc.get_sparse_core_info()
NC = _info.num_cores
NS = _info.num_subcores
NW = NC * NS

B_PER_W = 1560
CHUNK = 40
NCHUNK = B_PER_W // CHUNK
TAIL_BASE = NW * B_PER_W
TAIL_CHUNKS = (N_NODES - TAIL_BASE) // CHUNK
ROWS_PER_SUB = 7


def _sc_body(z_hbm, elec_hbm, w_hbm, out_hbm,
             t_sh, w_v, e_v, trow_v, zloc_v, buf0_v, buf1_v,
             gsem0, gsem1, wsem0, wsem1):
    c = lax.axis_index("c")
    s = lax.axis_index("s")
    wid = s * NC + c

    pltpu.sync_copy(w_hbm, w_v)
    pltpu.sync_copy(elec_hbm, e_v)
    lanes = lax.iota(jnp.int32, 16)

    def build_row(r, carry):
        zv = s * ROWS_PER_SUB + r

        @pl.when(zv < NZROWS)
        def _():
            for k in range(ROW // 16):
                o_blk, d_blk = divmod(16 * k, EMBED)
                f = 16 * k + lanes
                o = lax.shift_right_logical(f, 5)
                d = lax.bitwise_and(f, 31)
                e = plsc.load_gather(e_v, [zv * N_ORB + o])
                val = plsc.load_gather(w_v, [o * (W_ROWS * EMBED) + e * EMBED + d])
                val = jnp.where(e == 0, 0.0, val)
                trow_v[o_blk, pl.ds(d_blk, 16)] = val
            pltpu.sync_copy(trow_v, t_sh.at[zv])

        return carry

    lax.fori_loop(0, ROWS_PER_SUB, build_row, 0)
    plsc.subcore_barrier()

    base = wid * B_PER_W
    pltpu.sync_copy(z_hbm.at[pl.ds(base, B_PER_W)], zloc_v)

    def g_start(g, buf, sem):
        pltpu.async_copy(t_sh.at[zloc_v.at[pl.ds(g * CHUNK, CHUNK)]], buf, sem)

    def g_wait(buf, sem):
        pltpu.make_async_copy(
            t_sh.at[zloc_v.at[pl.ds(0, CHUNK)]], buf, sem).wait()

    def w_start(g, buf, sem):
        pltpu.async_copy(buf, out_hbm.at[pl.ds(base + g * CHUNK, CHUNK)], sem)

    def w_wait(buf, sem):
        pltpu.make_async_copy(buf, out_hbm.at[pl.ds(base, CHUNK)], sem).wait()

    g_start(0, buf0_v, gsem0)

    def pipe(p, carry):
        a = 2 * p
        b = a + 1
        g_wait(buf0_v, gsem0)

        @pl.when(b < NCHUNK)
        def _():
            @pl.when(p > 0)
            def _():
                w_wait(buf1_v, wsem1)
            g_start(b, buf1_v, gsem1)

        w_start(a, buf0_v, wsem0)

        @pl.when(b < NCHUNK)
        def _():
            g_wait(buf1_v, gsem1)

            @pl.when(b + 1 < NCHUNK)
            def _():
                w_wait(buf0_v, wsem0)
                g_start(b + 1, buf0_v, gsem0)

            w_start(b, buf1_v, wsem1)

        return carry

    lax.fori_loop(0, (NCHUNK + 1) // 2, pipe, 0)

    w_wait(buf0_v, wsem0)
    w_wait(buf1_v, wsem1)

    @pl.when(wid < TAIL_CHUNKS)
    def _():
        tb = TAIL_BASE + wid * CHUNK
        pltpu.sync_copy(z_hbm.at[pl.ds(tb, CHUNK)], zloc_v.at[pl.ds(0, CHUNK)])
        idx = zloc_v.at[pl.ds(0, CHUNK)]
        pltpu.async_copy(t_sh.at[idx], buf0_v, gsem0).wait()
        pltpu.sync_copy(buf0_v, out_hbm.at[pl.ds(tb, CHUNK)])


@jax.jit
def _run(z, elec_flat, w_flat):
    mesh = plsc.VectorSubcoreMesh(core_axis_name="c", subcore_axis_name="s")
    f = pl.kernel(
        _sc_body,
        out_type=jax.ShapeDtypeStruct((N_NODES, N_ORB, EMBED), jnp.float32),
        mesh=mesh,
        compiler_params=pltpu.CompilerParams(
            needs_layout_passes=False, use_tc_tiling_on_sc=False),
        scratch_types=[
            pltpu.VMEM_SHARED((NZROWS, N_ORB, EMBED), jnp.float32),
            pltpu.VMEM((W_FLAT,), jnp.float32),
            pltpu.VMEM((ELEC_PAD,), jnp.int32),
            pltpu.VMEM((N_ORB, EMBED), jnp.float32),
            pltpu.VMEM((B_PER_W,), jnp.int32),
            pltpu.VMEM((CHUNK, N_ORB, EMBED), jnp.float32),
            pltpu.VMEM((CHUNK, N_ORB, EMBED), jnp.float32),
            pltpu.SemaphoreType.DMA,
            pltpu.SemaphoreType.DMA,
            pltpu.SemaphoreType.DMA,
            pltpu.SemaphoreType.DMA,
        ],
    )
    return f(z, elec_flat, w_flat)


def kernel(z, elec_table, W):
    elec_flat = jnp.zeros((ELEC_PAD,), jnp.int32).at[: NZROWS * N_ORB].set(
        elec_table.reshape(-1))
    return _run(z, elec_flat, W.reshape(-1))


# speedup vs baseline: 20.9391x; 1.0252x over previous
import functools

import jax
import jax.numpy as jnp
from jax import lax
from jax.experimental import pallas as pl
from jax.experimental.pallas import tpu as pltpu
from jax.experimental.pallas import tpu_sc as plsc

N_ORB = 37
EMBED = 32
NZROWS = 97
W_ROWS = 16
W_FLAT = N_ORB * W_ROWS * EMBED
ELEC_PAD = 3600
N_NODES = 50000

_info = plsc.get_sparse_core_info()
NC = _info.num_cores
NS = _info.num_subcores
NW = NC * NS

MAX_ROWS_PER_W = 4
SEG = 4096
NSEG = N_NODES // SEG
TAIL = N_NODES - NSEG * SEG
MBUF = SEG + 16


def _sc_body(z_hbm, elec_hbm, w_hbm, out_hbm,
             rows_v, w_v, e_v, zseg_v, mn_v, mz_v, wsem):
    c = lax.axis_index("c")
    s = lax.axis_index("s")
    wid = s * NC + c
    lo = jnp.where(wid == 0, 0, 3 * wid + 1)
    hi = 3 * wid + 4

    pltpu.sync_copy(w_hbm, w_v)
    pltpu.sync_copy(elec_hbm, e_v)
    lanes = lax.iota(jnp.int32, 16)

    for r in range(MAX_ROWS_PER_W):
        zv = lo + r

        @pl.when(zv < hi)
        def _():
            for k in range((N_ORB * EMBED) // 16):
                o_blk, d_blk = divmod(16 * k, EMBED)
                f = 16 * k + lanes
                o = lax.shift_right_logical(f, 5)
                d = lax.bitwise_and(f, 31)
                e = plsc.load_gather(e_v, [zv * N_ORB + o])
                val = plsc.load_gather(w_v, [o * (W_ROWS * EMBED) + e * EMBED + d])
                val = jnp.where(e == 0, 0.0, val)
                rows_v[r * N_ORB + o_blk, pl.ds(d_blk, 16)] = val

    def row_wait():
        pltpu.make_async_copy(rows_v.at[pl.ds(0, N_ORB)],
                              out_hbm.at[0], wsem).wait()

    def do_segment(seg_base, seg_len):
        nchunk = seg_len // 16
        pltpu.sync_copy(z_hbm.at[pl.ds(seg_base, seg_len)],
                        zseg_v.at[pl.ds(0, seg_len)])

        def collect(i, cnt):
            zvec = zseg_v[pl.ds(i * 16, 16)]
            nvec = seg_base + i * 16 + lanes
            m = (zvec >= lo) & (zvec < hi)
            plsc.store_compressed(mn_v.at[pl.ds(cnt, 16)], nvec, mask=m)
            plsc.store_compressed(mz_v.at[pl.ds(cnt, 16)], zvec, mask=m)
            nhit = plsc.all_reduce_population_count(m)[0]
            return cnt + nhit

        cnt = lax.fori_loop(0, nchunk, collect, jnp.int32(0))

        def fire(k, carry):
            nvec = mn_v[pl.ds(k * 16, 16)]
            zvec = mz_v[pl.ds(k * 16, 16)]
            for j in range(16):
                @pl.when(k * 16 + j < cnt)
                def _():
                    pltpu.async_copy(
                        rows_v.at[pl.ds((zvec[j] - lo) * N_ORB, N_ORB)],
                        out_hbm.at[nvec[j]], wsem)
            for j in range(16):
                @pl.when(k * 16 + j < cnt)
                def _():
                    row_wait()
            return carry

        lax.fori_loop(0, (cnt + 15) // 16, fire, 0)

    lax.fori_loop(0, NSEG, lambda si, c: (do_segment(si * SEG, SEG), c)[1], 0)
    do_segment(NSEG * SEG, TAIL)


@jax.jit
def _run(z, elec_flat, w_flat):
    mesh = plsc.VectorSubcoreMesh(core_axis_name="c", subcore_axis_name="s")
    f = pl.kernel(
        _sc_body,
        out_type=jax.ShapeDtypeStruct((N_NODES, N_ORB, EMBED), jnp.float32),
        mesh=mesh,
        compiler_params=pltpu.CompilerParams(
            needs_layout_passes=False, use_tc_tiling_on_sc=True),
        scratch_types=[
            pltpu.VMEM((MAX_ROWS_PER_W * N_ORB, EMBED), jnp.float32),
            pltpu.VMEM((W_FLAT,), jnp.float32),
            pltpu.VMEM((ELEC_PAD,), jnp.int32),
            pltpu.VMEM((SEG,), jnp.int32),
            pltpu.VMEM((MBUF,), jnp.int32),
            pltpu.VMEM((MBUF,), jnp.int32),
            pltpu.SemaphoreType.DMA,
        ],
    )
    return f(z, elec_flat, w_flat)


def kernel(z, elec_table, W):
    elec_flat = jnp.zeros((ELEC_PAD,), jnp.int32).at[: NZROWS * N_ORB].set(
        elec_table.reshape(-1))
    return _run(z, elec_flat, W.reshape(-1))
